# Initial kernel scaffold; baseline (speedup 1.0000x reference)
import jax
import jax.numpy as jnp
from jax.experimental import pallas as pl


def kernel(spatial_info, entity_embeddings, locations, W_proj, b_proj):
    raise NotImplementedError("write your pallas kernel here")


# speedup vs baseline: 3.2235x; 3.2235x over previous
import jax
import jax.numpy as jnp
from jax import lax
from jax.experimental import pallas as pl
from jax.experimental.pallas import tpu as pltpu
from jax.experimental.pallas import tpu_sc as plsc

_B, _C, _H, _W = 16, 20, 128, 128
_N, _DIN, _D = 512, 256, 32
_HW = _H * _W
_CO = _C + _D
_TOTAL = _B * _CO * _HW
_NW = 32
_CHUNK = 128
_CPT = _B * _N * _D // (_NW * _CHUNK)
_GROUP = 16


def _tc_body(spatial_ref, emb_ref, lh_ref, lw_ref, w_ref, b_ref,
             out_ref, data_ref, idx_ref):
    b = pl.program_id(0)
    out_ref[0, :_C] = spatial_ref[0]
    out_ref[0, _C:] = jnp.zeros((_D, _H, _W), jnp.float32)
    proj = jnp.dot(emb_ref[0], w_ref[...], preferred_element_type=jnp.float32)
    proj = jnp.maximum(proj + b_ref[0, 0][None, :], 0.0)
    lh = jnp.clip(lh_ref[0, 0], 0, _H - 1)
    lw = jnp.clip(lw_ref[0, 0], 0, _W - 1)
    p = lh * _W + lw
    same = p[:, None] == p[None, :]
    col = lax.broadcasted_iota(jnp.int32, (_N, _N), 1)
    row = lax.broadcasted_iota(jnp.int32, (_N, _N), 0)
    has_later = jnp.any(same & (col > row), axis=1)
    sel = jnp.where(same & ~has_later[None, :], 1.0, 0.0)
    data_ref[0] = jnp.dot(sel, proj, preferred_element_type=jnp.float32)
    base = (b * _CO + _C) * _HW
    idx_ref[0] = (base + p)[:, None] + \
        lax.broadcasted_iota(jnp.int32, (_N, _D), 1) * _HW


def _sc_body(data_hbm, idx_hbm, out_hbm, idx_v, data_v, sem):
    wid = lax.axis_index("s") * 2 + lax.axis_index("c")
    pltpu.sync_copy(idx_hbm.at[wid], idx_v)
    pltpu.sync_copy(data_hbm.at[wid], data_v)

    @pl.loop(0, _CPT // _GROUP)
    def _wave(g):
        base = g * _GROUP
        copies = [
            pltpu.async_copy(data_v.at[base + j],
                             out_hbm.at[idx_v.at[base + j]], sem)
            for j in range(_GROUP)
        ]
        for cp in copies:
            cp.wait()


def _make_sc_scatter():
    return pl.kernel(
        _sc_body,
        out_type=(),
        mesh=plsc.VectorSubcoreMesh(core_axis_name="c", subcore_axis_name="s"),
        scratch_types=[
            pltpu.VMEM((_CPT, _CHUNK), jnp.int32),
            pltpu.VMEM((_CPT, _CHUNK), jnp.float32),
            pltpu.SemaphoreType.DMA,
        ],
    )


def kernel(spatial_info, entity_embeddings, locations, W_proj, b_proj):
    lh = locations[..., 0].reshape(_B, 1, _N)
    lw = locations[..., 1].reshape(_B, 1, _N)
    b3 = b_proj.reshape(1, 1, _D)
    out0, data, idxw = pl.pallas_call(
        _tc_body,
        grid=(_B,),
        in_specs=[
            pl.BlockSpec((1, _C, _H, _W), lambda b: (b, 0, 0, 0)),
            pl.BlockSpec((1, _N, _DIN), lambda b: (b, 0, 0)),
            pl.BlockSpec((1, 1, _N), lambda b: (b, 0, 0)),
            pl.BlockSpec((1, 1, _N), lambda b: (b, 0, 0)),
            pl.BlockSpec((_DIN, _D), lambda b: (0, 0)),
            pl.BlockSpec((1, 1, _D), lambda b: (0, 0, 0)),
        ],
        out_specs=[
            pl.BlockSpec((1, _CO, _H, _W), lambda b: (b, 0, 0, 0)),
            pl.BlockSpec((1, _N, _D), lambda b: (b, 0, 0)),
            pl.BlockSpec((1, _N, _D), lambda b: (b, 0, 0)),
        ],
        out_shape=[
            jax.ShapeDtypeStruct((_B, _CO, _H, _W), jnp.float32),
            jax.ShapeDtypeStruct((_B, _N, _D), jnp.float32),
            jax.ShapeDtypeStruct((_B, _N, _D), jnp.int32),
        ],
    )(spatial_info, entity_embeddings, lh, lw, W_proj, b3)
    data_t = data.reshape(_NW, _CPT, _CHUNK)
    idx_t = idxw.reshape(_NW, _CPT, _CHUNK)
    out_ref = jax.new_ref(out0.reshape(_TOTAL))
    _make_sc_scatter()(data_t, idx_t, out_ref)
    return jax.freeze(out_ref).reshape(_B, _CO, _H, _W)


# speedup vs baseline: 3.2353x; 1.0036x over previous
import jax
import jax.numpy as jnp
from jax import lax
from jax.experimental import pallas as pl
from jax.experimental.pallas import tpu as pltpu
from jax.experimental.pallas import tpu_sc as plsc

_B, _C, _H, _W = 16, 20, 128, 128
_N, _DIN, _D = 512, 256, 32
_HW = _H * _W
_CO = _C + _D
_TOTAL = _B * _CO * _HW
_NW = 32
_CHUNK = 128
_CPT = _B * _N * _D // (_NW * _CHUNK)
_GROUP = 16


def _tc_body(spatial_ref, emb_ref, lh_ref, lw_ref, w_ref, b_ref,
             out_ref, data_ref, idx_ref):
    b = pl.program_id(0)
    out_ref[0, :_C] = spatial_ref[0]
    out_ref[0, _C:] = jnp.zeros((_D, _H, _W), jnp.float32)
    proj = jnp.dot(emb_ref[0], w_ref[...], preferred_element_type=jnp.float32)
    proj = jnp.maximum(proj + b_ref[0, 0][None, :], 0.0)
    lh = jnp.clip(lh_ref[0, 0], 0, _H - 1)
    lw = jnp.clip(lw_ref[0, 0], 0, _W - 1)
    p = lh * _W + lw
    same = p[:, None] == p[None, :]
    col = lax.broadcasted_iota(jnp.int32, (_N, _N), 1)
    row = lax.broadcasted_iota(jnp.int32, (_N, _N), 0)
    has_later = jnp.any(same & (col > row), axis=1)
    sel = jnp.where(same & ~has_later[None, :], 1.0, 0.0)
    data_ref[0] = jnp.dot(sel, proj, preferred_element_type=jnp.float32)
    base = (b * _CO + _C) * _HW
    idx_ref[0] = (base + p)[:, None] + \
        lax.broadcasted_iota(jnp.int32, (_N, _D), 1) * _HW


def _sc_body(data_hbm, idx_hbm, out_hbm, idx_v, data_v, sem):
    wid = lax.axis_index("s") * 2 + lax.axis_index("c")
    pltpu.sync_copy(idx_hbm.at[wid], idx_v)
    pltpu.sync_copy(data_hbm.at[wid], data_v)

    @pl.loop(0, _CPT)
    def _issue(j):
        pltpu.async_copy(data_v.at[j], out_hbm.at[idx_v.at[j]], sem)

    pltpu.make_async_copy(data_hbm.at[wid], data_v, sem).wait()


def _make_sc_scatter():
    return pl.kernel(
        _sc_body,
        out_type=(),
        mesh=plsc.VectorSubcoreMesh(core_axis_name="c", subcore_axis_name="s"),
        scratch_types=[
            pltpu.VMEM((_CPT, _CHUNK), jnp.int32),
            pltpu.VMEM((_CPT, _CHUNK), jnp.float32),
            pltpu.SemaphoreType.DMA,
        ],
    )


def kernel(spatial_info, entity_embeddings, locations, W_proj, b_proj):
    lh = locations[..., 0].reshape(_B, 1, _N)
    lw = locations[..., 1].reshape(_B, 1, _N)
    b3 = b_proj.reshape(1, 1, _D)
    out0, data, idxw = pl.pallas_call(
        _tc_body,
        grid=(_B,),
        in_specs=[
            pl.BlockSpec((1, _C, _H, _W), lambda b: (b, 0, 0, 0)),
            pl.BlockSpec((1, _N, _DIN), lambda b: (b, 0, 0)),
            pl.BlockSpec((1, 1, _N), lambda b: (b, 0, 0)),
            pl.BlockSpec((1, 1, _N), lambda b: (b, 0, 0)),
            pl.BlockSpec((_DIN, _D), lambda b: (0, 0)),
            pl.BlockSpec((1, 1, _D), lambda b: (0, 0, 0)),
        ],
        out_specs=[
            pl.BlockSpec((1, _CO, _H, _W), lambda b: (b, 0, 0, 0)),
            pl.BlockSpec((1, _N, _D), lambda b: (b, 0, 0)),
            pl.BlockSpec((1, _N, _D), lambda b: (b, 0, 0)),
        ],
        out_shape=[
            jax.ShapeDtypeStruct((_B, _CO, _H, _W), jnp.float32),
            jax.ShapeDtypeStruct((_B, _N, _D), jnp.float32),
            jax.ShapeDtypeStruct((_B, _N, _D), jnp.int32),
        ],
    )(spatial_info, entity_embeddings, lh, lw, W_proj, b3)
    data_t = data.reshape(_NW, _CPT, _CHUNK)
    idx_t = idxw.reshape(_NW, _CPT, _CHUNK)
    out_ref = jax.new_ref(out0.reshape(_TOTAL))
    _make_sc_scatter()(data_t, idx_t, out_ref)
    return jax.freeze(out_ref).reshape(_B, _CO, _H, _W)


# speedup vs baseline: 9.2141x; 2.8480x over previous
import jax
import jax.numpy as jnp
from jax import lax
from jax.experimental import pallas as pl
from jax.experimental.pallas import tpu as pltpu
from jax.experimental.pallas import tpu_sc as plsc

_B, _C, _H, _W = 16, 20, 128, 128
_N, _DIN, _D = 512, 256, 32
_HW = _H * _W
_CO = _C + _D
_TOTAL = _B * _CO * _HW
_PLANE = _D * _HW
_NT = 16
_SLAB = _PLANE // _NT
_WPB = _N * _D
_WPT = _WPB // _NT
_NSTR = _WPT // 128
_ZB = 2048
_BPC = _B // 2


def _tc_body(spatial_ref, emb_ref, lh_ref, lw_ref, w_ref, b_ref,
             out_ref, data_ref, idx_ref):
    out_ref[0, :_C] = spatial_ref[0]
    proj = jnp.dot(emb_ref[0], w_ref[...], preferred_element_type=jnp.float32)
    proj = jnp.maximum(proj + b_ref[0, 0][None, :], 0.0)
    lh = jnp.clip(lh_ref[0, 0], 0, _H - 1)
    lw = jnp.clip(lw_ref[0, 0], 0, _W - 1)
    p = lh * _W + lw
    same = p[:, None] == p[None, :]
    col = lax.broadcasted_iota(jnp.int32, (_N, _N), 1)
    row = lax.broadcasted_iota(jnp.int32, (_N, _N), 0)
    has_later = jnp.any(same & (col > row), axis=1)
    sel = jnp.where(same & ~has_later[None, :], 1.0, 0.0)
    data_ref[0] = jnp.dot(sel, proj, preferred_element_type=jnp.float32)
    idx_ref[0] = p[:, None] + \
        lax.broadcasted_iota(jnp.int32, (_N, _D), 1) * _HW


def _sc_body(data_hbm, idx_hbm, out_hbm, plane, idx_v, data_v, zbig):
    c = lax.axis_index("c")
    t = lax.axis_index("s")

    @pl.loop(0, _ZB // 16)
    def _zfill(i):
        zbig[pl.ds(i * 16, 16)] = jnp.zeros((16,), jnp.float32)

    @pl.loop(0, _SLAB // _ZB)
    def _zslab(i):
        pltpu.sync_copy(zbig, plane.at[pl.ds(t * _SLAB + i * _ZB, _ZB)])

    plsc.subcore_barrier()

    @pl.loop(0, _BPC)
    def _batch(bl):
        b = c * _BPC + bl
        pltpu.sync_copy(idx_hbm.at[b * _NT + t], idx_v)
        pltpu.sync_copy(data_hbm.at[b * _NT + t], data_v)
        for j in range(_NSTR):
            pltpu.sync_copy(data_v.at[j], plane.at[idx_v.at[j]])
        plsc.subcore_barrier()
        pltpu.sync_copy(
            plane.at[pl.ds(t * _SLAB, _SLAB)],
            out_hbm.at[pl.ds((b * _CO + _C) * _HW + t * _SLAB, _SLAB)])
        plsc.subcore_barrier()
        for j in range(_NSTR):
            pltpu.sync_copy(zbig.at[pl.ds(0, 128)], plane.at[idx_v.at[j]])
        plsc.subcore_barrier()


def _make_sc_scatter():
    return pl.kernel(
        _sc_body,
        out_type=(),
        mesh=plsc.VectorSubcoreMesh(core_axis_name="c", subcore_axis_name="s"),
        scratch_types=[
            pltpu.VMEM_SHARED((_PLANE,), jnp.float32),
            pltpu.VMEM((_NSTR, 128), jnp.int32),
            pltpu.VMEM((_NSTR, 128), jnp.float32),
            pltpu.VMEM((_ZB,), jnp.float32),
        ],
    )


def kernel(spatial_info, entity_embeddings, locations, W_proj, b_proj):
    lh = locations[..., 0].reshape(_B, 1, _N)
    lw = locations[..., 1].reshape(_B, 1, _N)
    b3 = b_proj.reshape(1, 1, _D)
    out0, data, idxw = pl.pallas_call(
        _tc_body,
        grid=(_B,),
        in_specs=[
            pl.BlockSpec((1, _C, _H, _W), lambda b: (b, 0, 0, 0)),
            pl.BlockSpec((1, _N, _DIN), lambda b: (b, 0, 0)),
            pl.BlockSpec((1, 1, _N), lambda b: (b, 0, 0)),
            pl.BlockSpec((1, 1, _N), lambda b: (b, 0, 0)),
            pl.BlockSpec((_DIN, _D), lambda b: (0, 0)),
            pl.BlockSpec((1, 1, _D), lambda b: (0, 0, 0)),
        ],
        out_specs=[
            pl.BlockSpec((1, _CO, _H, _W), lambda b: (b, 0, 0, 0)),
            pl.BlockSpec((1, _N, _D), lambda b: (b, 0, 0)),
            pl.BlockSpec((1, _N, _D), lambda b: (b, 0, 0)),
        ],
        out_shape=[
            jax.ShapeDtypeStruct((_B, _CO, _H, _W), jnp.float32),
            jax.ShapeDtypeStruct((_B, _N, _D), jnp.float32),
            jax.ShapeDtypeStruct((_B, _N, _D), jnp.int32),
        ],
    )(spatial_info, entity_embeddings, lh, lw, W_proj, b3)
    data_t = data.reshape(_B * _NT, _NSTR, 128)
    idx_t = idxw.reshape(_B * _NT, _NSTR, 128)
    out_ref = jax.new_ref(out0.reshape(_TOTAL))
    _make_sc_scatter()(data_t, idx_t, out_ref)
    return jax.freeze(out_ref).reshape(_B, _CO, _H, _W)


# speedup vs baseline: 9.8511x; 1.0691x over previous
import jax
import jax.numpy as jnp
from jax import lax
from jax.experimental import pallas as pl
from jax.experimental.pallas import tpu as pltpu
from jax.experimental.pallas import tpu_sc as plsc

_B, _C, _H, _W = 16, 20, 128, 128
_N, _DIN, _D = 512, 256, 32
_HW = _H * _W
_CO = _C + _D
_TOTAL = _B * _CO * _HW
_PLANE = _D * _HW
_NT = 16
_SLAB = _PLANE // _NT
_WPB = _N * _D
_WPT = _WPB // _NT
_NSTR = _WPT // 128
_ZB = 2048
_BPC = _B // 2


def _tc_body(spatial_ref, emb_ref, lh_ref, lw_ref, w_ref, b_ref,
             out_ref, data_ref, idx_ref):
    out_ref[0] = spatial_ref[0]
    proj = jnp.dot(emb_ref[0], w_ref[...], preferred_element_type=jnp.float32)
    proj = jnp.maximum(proj + b_ref[0, 0][None, :], 0.0)
    lh = jnp.clip(lh_ref[0, 0], 0, _H - 1)
    lw = jnp.clip(lw_ref[0, 0], 0, _W - 1)
    p = lh * _W + lw
    same = p[:, None] == p[None, :]
    col = lax.broadcasted_iota(jnp.int32, (_N, _N), 1)
    row = lax.broadcasted_iota(jnp.int32, (_N, _N), 0)
    has_later = jnp.any(same & (col > row), axis=1)
    sel = jnp.where(same & ~has_later[None, :], 1.0, 0.0)
    data_ref[0] = jnp.dot(sel, proj, preferred_element_type=jnp.float32)
    idx_ref[0] = p[:, None] + \
        lax.broadcasted_iota(jnp.int32, (_N, _D), 1) * _HW


def _sc_body(data_hbm, idx_hbm, out_hbm, plane, idx_v, data_v, zbig):
    c = lax.axis_index("c")
    t = lax.axis_index("s")

    @pl.loop(0, _ZB // 16)
    def _zfill(i):
        zbig[pl.ds(i * 16, 16)] = jnp.zeros((16,), jnp.float32)

    @pl.loop(0, _SLAB // _ZB)
    def _zslab(i):
        pltpu.sync_copy(zbig, plane.at[pl.ds(t * _SLAB + i * _ZB, _ZB)])

    plsc.subcore_barrier()

    @pl.loop(0, _BPC)
    def _batch(bl):
        b = c * _BPC + bl
        pltpu.sync_copy(idx_hbm.at[b * _NT + t], idx_v)
        pltpu.sync_copy(data_hbm.at[b * _NT + t], data_v)
        for j in range(_NSTR):
            pltpu.sync_copy(data_v.at[j], plane.at[idx_v.at[j]])
        plsc.subcore_barrier()
        pltpu.sync_copy(
            plane.at[pl.ds(t * _SLAB, _SLAB)],
            out_hbm.at[pl.ds((b * _CO + _C) * _HW + t * _SLAB, _SLAB)])
        plsc.subcore_barrier()
        for j in range(_NSTR):
            pltpu.sync_copy(zbig.at[pl.ds(0, 128)], plane.at[idx_v.at[j]])
        plsc.subcore_barrier()


def _make_sc_scatter():
    return pl.kernel(
        _sc_body,
        out_type=(),
        mesh=plsc.VectorSubcoreMesh(core_axis_name="c", subcore_axis_name="s"),
        scratch_types=[
            pltpu.VMEM_SHARED((_PLANE,), jnp.float32),
            pltpu.VMEM((_NSTR, 128), jnp.int32),
            pltpu.VMEM((_NSTR, 128), jnp.float32),
            pltpu.VMEM((_ZB,), jnp.float32),
        ],
    )


def kernel(spatial_info, entity_embeddings, locations, W_proj, b_proj):
    lh = locations[..., 0].reshape(_B, 1, _N)
    lw = locations[..., 1].reshape(_B, 1, _N)
    b3 = b_proj.reshape(1, 1, _D)
    out0, data, idxw = pl.pallas_call(
        _tc_body,
        grid=(_B,),
        in_specs=[
            pl.BlockSpec((1, _C, _H, _W), lambda b: (b, 0, 0, 0)),
            pl.BlockSpec((1, _N, _DIN), lambda b: (b, 0, 0)),
            pl.BlockSpec((1, 1, _N), lambda b: (b, 0, 0)),
            pl.BlockSpec((1, 1, _N), lambda b: (b, 0, 0)),
            pl.BlockSpec((_DIN, _D), lambda b: (0, 0)),
            pl.BlockSpec((1, 1, _D), lambda b: (0, 0, 0)),
        ],
        out_specs=[
            pl.BlockSpec((1, _C, _H, _W), lambda b: (b, 0, 0, 0)),
            pl.BlockSpec((1, _N, _D), lambda b: (b, 0, 0)),
            pl.BlockSpec((1, _N, _D), lambda b: (b, 0, 0)),
        ],
        out_shape=[
            jax.ShapeDtypeStruct((_B, _CO, _H, _W), jnp.float32),
            jax.ShapeDtypeStruct((_B, _N, _D), jnp.float32),
            jax.ShapeDtypeStruct((_B, _N, _D), jnp.int32),
        ],
    )(spatial_info, entity_embeddings, lh, lw, W_proj, b3)
    data_t = data.reshape(_B * _NT, _NSTR, 128)
    idx_t = idxw.reshape(_B * _NT, _NSTR, 128)
    out_ref = jax.new_ref(out0.reshape(_TOTAL))
    _make_sc_scatter()(data_t, idx_t, out_ref)
    return jax.freeze(out_ref).reshape(_B, _CO, _H, _W)
